# baseline (device time: 14993 ns/iter reference)
import jax
import jax.numpy as jnp
from jax import lax
from jax.experimental import pallas as pl
from jax.experimental.pallas import tpu as pltpu

N_DEV = 8
N_EXPERTS = 16
CAPACITY = 25
CAP_PAD = 32


def kernel(x, router_W, route_idx, expert_W):
    n_tok, d_in = x.shape
    e_per, _, d_out = expert_W.shape
    blk = e_per * CAP_PAD
    n_slots = N_EXPERTS * CAP_PAD

    def body(x_hbm, idx_hbm, w_hbm, out_hbm,
             x_v, idx_v, w_v, out_v, gbuf, copy_sems, send_sems, recv_sems):
        my = lax.axis_index("i")

        barrier_sem = pltpu.get_barrier_semaphore()
        for m in range(1, N_DEV):
            pl.semaphore_signal(
                barrier_sem, inc=1,
                device_id=(my ^ m,), device_id_type=pl.DeviceIdType.MESH,
            )

        cp_idx = pltpu.make_async_copy(idx_hbm, idx_v, copy_sems.at[0])
        cp_x = pltpu.make_async_copy(x_hbm, x_v, copy_sems.at[1])
        cp_w = pltpu.make_async_copy(w_hbm, w_v, copy_sems.at[2])
        cp_idx.start()
        cp_x.start()
        cp_w.start()
        cp_idx.wait()

        idx = idx_v[:, :]
        eids = lax.broadcasted_iota(jnp.int32, (n_tok, N_EXPERTS), 1)
        onehot = (idx == eids).astype(jnp.float32)
        row = lax.broadcasted_iota(jnp.int32, (n_tok, n_tok), 0)
        col = lax.broadcasted_iota(jnp.int32, (n_tok, n_tok), 1)
        tril = (col <= row).astype(jnp.float32)
        cum = jnp.dot(tril, onehot, preferred_element_type=jnp.float32)
        pos = jnp.sum(cum * onehot, axis=1, keepdims=True)
        pos_i = pos.astype(jnp.int32)

        slot_t = jnp.where(
            pos_i <= CAPACITY, idx * CAP_PAD + pos_i - 1, -1
        )

        lsl = slot_t - my * blk
        P_local_t = (
            lax.broadcasted_iota(jnp.int32, (n_tok, blk), 1) == lsl
        ).astype(jnp.float32)
        cp_x.wait()
        cx = lax.dot_general(
            P_local_t, x_v[:, :], (((0,), (0,)), ((), ())),
            preferred_element_type=jnp.float32,
        )
        cp_w.wait()
        y0 = jnp.dot(cx[:CAP_PAD], w_v[0, :, :],
                     preferred_element_type=jnp.float32)
        y1 = jnp.dot(cx[CAP_PAD:], w_v[1, :, :],
                     preferred_element_type=jnp.float32)
        gbuf[pl.ds(my, 1)] = (
            jnp.concatenate([y0, y1], axis=0)
            .astype(jnp.bfloat16)
            .reshape(1, blk, d_out)
        )

        pl.semaphore_wait(barrier_sem, N_DEV - 1)

        rdmas = []
        for m in range(1, N_DEV):
            rdma = pltpu.make_async_remote_copy(
                src_ref=gbuf.at[pl.ds(my, 1)],
                dst_ref=gbuf.at[pl.ds(my, 1)],
                send_sem=send_sems.at[m - 1],
                recv_sem=recv_sems.at[m - 1],
                device_id=(my ^ m,),
                device_id_type=pl.DeviceIdType.MESH,
            )
            rdma.start()
            rdmas.append(rdma)

        s_id = lax.broadcasted_iota(jnp.int32, (n_tok, n_slots), 1)
        Pt = (s_id == slot_t).astype(jnp.float32).astype(jnp.bfloat16)

        for rdma in rdmas:
            rdma.wait()

        y_all = gbuf[...].reshape(n_slots, d_out)
        out_v[:, :] = jnp.dot(Pt, y_all, preferred_element_type=jnp.float32)
        cp_out = pltpu.make_async_copy(out_v, out_hbm, copy_sems.at[3])
        cp_out.start()
        cp_out.wait()

    return pl.pallas_call(
        body,
        out_shape=jax.ShapeDtypeStruct((n_tok, d_out), jnp.float32),
        in_specs=[
            pl.BlockSpec(memory_space=pl.ANY),
            pl.BlockSpec(memory_space=pl.ANY),
            pl.BlockSpec(memory_space=pl.ANY),
        ],
        out_specs=pl.BlockSpec(memory_space=pl.ANY),
        scratch_shapes=[
            pltpu.VMEM((n_tok, d_in), jnp.float32),
            pltpu.VMEM((n_tok, 1), jnp.int32),
            pltpu.VMEM((e_per, d_in, d_out), jnp.float32),
            pltpu.VMEM((n_tok, d_out), jnp.float32),
            pltpu.VMEM((N_DEV, blk, d_out), jnp.bfloat16),
            pltpu.SemaphoreType.DMA((4,)),
            pltpu.SemaphoreType.DMA((N_DEV - 1,)),
            pltpu.SemaphoreType.DMA((N_DEV - 1,)),
        ],
        compiler_params=pltpu.CompilerParams(collective_id=0),
    )(x, route_idx, expert_W)


# device time: 14697 ns/iter; 1.0201x vs baseline; 1.0201x over previous
import jax
import jax.numpy as jnp
from jax import lax
from jax.experimental import pallas as pl
from jax.experimental.pallas import tpu as pltpu

N_DEV = 8
N_EXPERTS = 16
CAPACITY = 25
CAP_PAD = 25


def kernel(x, router_W, route_idx, expert_W):
    n_tok, d_in = x.shape
    e_per, _, d_out = expert_W.shape
    blk = e_per * CAP_PAD
    n_slots = N_EXPERTS * CAP_PAD

    def body(x_hbm, idx_hbm, w_hbm, out_hbm,
             x_v, idx_v, w_v, out_v, gbuf, copy_sems, send_sems, recv_sems):
        my = lax.axis_index("i")

        barrier_sem = pltpu.get_barrier_semaphore()
        for m in range(1, N_DEV):
            pl.semaphore_signal(
                barrier_sem, inc=1,
                device_id=(my ^ m,), device_id_type=pl.DeviceIdType.MESH,
            )

        cp_idx = pltpu.make_async_copy(idx_hbm, idx_v, copy_sems.at[0])
        cp_x = pltpu.make_async_copy(x_hbm, x_v, copy_sems.at[1])
        cp_w = pltpu.make_async_copy(w_hbm, w_v, copy_sems.at[2])
        cp_idx.start()
        cp_x.start()
        cp_w.start()

        eids = lax.broadcasted_iota(jnp.int32, (n_tok, N_EXPERTS), 1)
        row = lax.broadcasted_iota(jnp.int32, (n_tok, n_tok), 0)
        col = lax.broadcasted_iota(jnp.int32, (n_tok, n_tok), 1)
        tril = (col <= row).astype(jnp.float32)
        loc_id = lax.broadcasted_iota(jnp.int32, (n_tok, blk), 1)
        s_id = lax.broadcasted_iota(jnp.int32, (n_tok, n_slots), 1)
        cp_idx.wait()

        idx = idx_v[:, :]
        onehot = (idx == eids).astype(jnp.float32)
        cum = jnp.dot(tril, onehot, preferred_element_type=jnp.float32)
        pos = jnp.sum(cum * onehot, axis=1, keepdims=True)
        pos_i = pos.astype(jnp.int32)

        slot_t = jnp.where(
            pos_i <= CAPACITY, idx * CAP_PAD + pos_i - 1, -1
        )

        lsl = slot_t - my * blk
        P_local_t = (loc_id == lsl).astype(jnp.float32)
        cp_x.wait()
        cx = lax.dot_general(
            P_local_t, x_v[:, :], (((0,), (0,)), ((), ())),
            preferred_element_type=jnp.float32,
        )
        cp_w.wait()
        y0 = jnp.dot(cx[:CAP_PAD], w_v[0, :, :],
                     preferred_element_type=jnp.float32)
        y1 = jnp.dot(cx[CAP_PAD:], w_v[1, :, :],
                     preferred_element_type=jnp.float32)
        gbuf[pl.ds(my, 1)] = (
            jnp.concatenate([y0, y1], axis=0)
            .astype(jnp.bfloat16)
            .reshape(1, blk, d_out)
        )

        pl.semaphore_wait(barrier_sem, N_DEV - 1)

        rdmas = []
        for m in range(1, N_DEV):
            rdma = pltpu.make_async_remote_copy(
                src_ref=gbuf.at[pl.ds(my, 1)],
                dst_ref=gbuf.at[pl.ds(my, 1)],
                send_sem=send_sems.at[m - 1],
                recv_sem=recv_sems.at[m - 1],
                device_id=(my ^ m,),
                device_id_type=pl.DeviceIdType.MESH,
            )
            rdma.start()
            rdmas.append(rdma)

        Pt = (s_id == slot_t).astype(jnp.float32).astype(jnp.bfloat16)

        for rdma in rdmas:
            rdma.wait()

        y_all = gbuf[...].reshape(n_slots, d_out)
        out_v[:, :] = jnp.dot(Pt, y_all, preferred_element_type=jnp.float32)
        cp_out = pltpu.make_async_copy(out_v, out_hbm, copy_sems.at[3])
        cp_out.start()
        cp_out.wait()

    return pl.pallas_call(
        body,
        out_shape=jax.ShapeDtypeStruct((n_tok, d_out), jnp.float32),
        in_specs=[
            pl.BlockSpec(memory_space=pl.ANY),
            pl.BlockSpec(memory_space=pl.ANY),
            pl.BlockSpec(memory_space=pl.ANY),
        ],
        out_specs=pl.BlockSpec(memory_space=pl.ANY),
        scratch_shapes=[
            pltpu.VMEM((n_tok, d_in), jnp.float32),
            pltpu.VMEM((n_tok, 1), jnp.int32),
            pltpu.VMEM((e_per, d_in, d_out), jnp.float32),
            pltpu.VMEM((n_tok, d_out), jnp.float32),
            pltpu.VMEM((N_DEV, blk, d_out), jnp.bfloat16),
            pltpu.SemaphoreType.DMA((4,)),
            pltpu.SemaphoreType.DMA((N_DEV - 1,)),
            pltpu.SemaphoreType.DMA((N_DEV - 1,)),
        ],
        compiler_params=pltpu.CompilerParams(collective_id=0),
    )(x, route_idx, expert_W)


# device time: 14175 ns/iter; 1.0577x vs baseline; 1.0368x over previous
import jax
import jax.numpy as jnp
from jax import lax
from jax.experimental import pallas as pl
from jax.experimental.pallas import tpu as pltpu

N_DEV = 8
N_EXPERTS = 16
CAPACITY = 25
CAP_PAD = 25


def kernel(x, router_W, route_idx, expert_W):
    n_tok, d_in = x.shape
    e_per, _, d_out = expert_W.shape
    blk = e_per * CAP_PAD
    n_slots = N_EXPERTS * CAP_PAD

    def body(x_hbm, idx_hbm, w_hbm, out_hbm,
             x_v, idx_v, w_v, out_v, gbuf, copy_sems, send_sems, recv_sems):
        my = lax.axis_index("i")

        barrier_sem = pltpu.get_barrier_semaphore()
        for m in range(1, N_DEV):
            pl.semaphore_signal(
                barrier_sem, inc=1,
                device_id=(my ^ m,), device_id_type=pl.DeviceIdType.MESH,
            )

        cp_idx = pltpu.make_async_copy(idx_hbm, idx_v, copy_sems.at[0])
        cp_x = pltpu.make_async_copy(x_hbm, x_v, copy_sems.at[1])
        cp_w = pltpu.make_async_copy(w_hbm, w_v, copy_sems.at[2])
        cp_idx.start()
        cp_x.start()
        cp_w.start()

        eids = lax.broadcasted_iota(jnp.int32, (n_tok, N_EXPERTS), 1)
        row = lax.broadcasted_iota(jnp.int32, (n_tok, n_tok), 0)
        col = lax.broadcasted_iota(jnp.int32, (n_tok, n_tok), 1)
        tril = (col <= row).astype(jnp.float32)
        loc_id = lax.broadcasted_iota(jnp.int32, (n_tok, blk), 1)
        s_id = lax.broadcasted_iota(jnp.int32, (n_tok, n_slots), 1)
        cp_idx.wait()

        idx = idx_v[:, :]
        onehot = (idx == eids).astype(jnp.float32)
        cum = jnp.dot(tril, onehot, preferred_element_type=jnp.float32)
        pos = jnp.sum(cum * onehot, axis=1, keepdims=True)
        pos_i = pos.astype(jnp.int32)

        slot_t = jnp.where(
            pos_i <= CAPACITY, idx * CAP_PAD + pos_i - 1, -1
        )

        lsl = slot_t - my * blk
        P_local_t = (loc_id == lsl).astype(jnp.float32)
        cp_x.wait()
        cx = lax.dot_general(
            P_local_t, x_v[:, :], (((0,), (0,)), ((), ())),
            preferred_element_type=jnp.float32,
        )
        cp_w.wait()
        y0 = jnp.dot(cx[:CAP_PAD], w_v[0, :, :],
                     preferred_element_type=jnp.float32)
        y1 = jnp.dot(cx[CAP_PAD:], w_v[1, :, :],
                     preferred_element_type=jnp.float32)
        gbuf[pl.ds(my, 1)] = (
            jnp.concatenate([y0, y1], axis=0)
            .astype(jnp.bfloat16)
            .reshape(1, blk, d_out)
        )

        pl.semaphore_wait(barrier_sem, N_DEV - 1)

        NEAR = (1, 3, 4)
        FAR = (6, 2, 5, 7)
        rdmas = {}
        for m in FAR + NEAR:
            rdma = pltpu.make_async_remote_copy(
                src_ref=gbuf.at[pl.ds(my, 1)],
                dst_ref=gbuf.at[pl.ds(my, 1)],
                send_sem=send_sems.at[m - 1],
                recv_sem=recv_sems.at[m - 1],
                device_id=(my ^ m,),
                device_id_type=pl.DeviceIdType.MESH,
            )
            rdma.start()
            rdmas[m] = rdma

        def piece(p):
            return (loc_id == (slot_t - p * blk)).astype(
                jnp.float32
            ).astype(jnp.bfloat16)

        near_ids = [my] + [my ^ m for m in NEAR]
        far_ids = [my ^ m for m in FAR]
        Pt_near = jnp.concatenate([piece(p) for p in near_ids], axis=1)
        Pt_far = jnp.concatenate([piece(p) for p in far_ids], axis=1)

        for m in NEAR:
            rdmas[m].wait_recv()
        y_near = jnp.concatenate(
            [gbuf[pl.ds(p, 1)].reshape(blk, d_out) for p in near_ids],
            axis=0,
        )
        acc = jnp.dot(Pt_near, y_near, preferred_element_type=jnp.float32)

        for m in FAR:
            rdmas[m].wait_recv()
        y_far = jnp.concatenate(
            [gbuf[pl.ds(p, 1)].reshape(blk, d_out) for p in far_ids],
            axis=0,
        )
        out_v[:, :] = acc + jnp.dot(
            Pt_far, y_far, preferred_element_type=jnp.float32
        )
        cp_out = pltpu.make_async_copy(out_v, out_hbm, copy_sems.at[3])
        cp_out.start()

        for m in FAR + NEAR:
            rdmas[m].wait_send()
        cp_out.wait()

    return pl.pallas_call(
        body,
        out_shape=jax.ShapeDtypeStruct((n_tok, d_out), jnp.float32),
        in_specs=[
            pl.BlockSpec(memory_space=pl.ANY),
            pl.BlockSpec(memory_space=pl.ANY),
            pl.BlockSpec(memory_space=pl.ANY),
        ],
        out_specs=pl.BlockSpec(memory_space=pl.ANY),
        scratch_shapes=[
            pltpu.VMEM((n_tok, d_in), jnp.float32),
            pltpu.VMEM((n_tok, 1), jnp.int32),
            pltpu.VMEM((e_per, d_in, d_out), jnp.float32),
            pltpu.VMEM((n_tok, d_out), jnp.float32),
            pltpu.VMEM((N_DEV, blk, d_out), jnp.bfloat16),
            pltpu.SemaphoreType.DMA((4,)),
            pltpu.SemaphoreType.DMA((N_DEV - 1,)),
            pltpu.SemaphoreType.DMA((N_DEV - 1,)),
        ],
        compiler_params=pltpu.CompilerParams(collective_id=0),
    )(x, route_idx, expert_W)
